# baseline (speedup 1.0000x reference)
import functools

import jax
import jax.numpy as jnp
from jax import lax
from jax.experimental import pallas as pl
from jax.experimental.pallas import tpu as pltpu
from jax.experimental.pallas import tpu_sc as plsc

NUM_CORES = 2
NUM_SUBCORES = 16
NUM_WORKERS = NUM_CORES * NUM_SUBCORES
LANE = 128
GRP = 4


def _tc_detile(vocab: int, emb_dim: int):
    n_main = 999936 // GRP
    n_blk = n_main // LANE
    n_rows = vocab // GRP

    def body(i0, i1, i2, i3, tail_ref, out_ref):
        pid = pl.program_id(0)
        eye = jax.lax.broadcasted_iota(jnp.int32, (emb_dim, emb_dim), 0)
        eye = (eye == jax.lax.broadcasted_iota(
            jnp.int32, (emb_dim, emb_dim), 1)).astype(jnp.float32)

        @pl.when(pid < n_blk)
        def _():
            for k, ref in enumerate((i0, i1, i2, i3)):
                out_ref[:, k * emb_dim:(k + 1) * emb_dim] = (
                    jax.lax.dot_general(
                        ref[...], eye, (((0,), (0,)), ((), ())),
                        precision=jax.lax.Precision.HIGHEST,
                    )
                )

        @pl.when(pid == n_blk)
        def _():
            out_ref[0:(vocab - GRP * n_main) // GRP, :] = tail_ref[...]

    in_specs = [
        pl.BlockSpec((emb_dim, LANE), (lambda i, K=K: (0, i + K * n_blk)))
        for K in range(GRP)
    ]
    in_specs.append(
        pl.BlockSpec(((vocab - GRP * n_main) // GRP, LANE), lambda i: (0, 0))
    )
    return pl.pallas_call(
        body,
        grid=(n_blk + 1,),
        in_specs=in_specs,
        out_specs=pl.BlockSpec((LANE, GRP * emb_dim), lambda i: (i, 0)),
        out_shape=jax.ShapeDtypeStruct((n_rows, GRP * emb_dim), jnp.float32),
    )


def _emb_lookup(hist: int, emb_dim: int, n_bblk: int):
    n_dblk = emb_dim // 8
    mesh = plsc.VectorSubcoreMesh(core_axis_name="c", subcore_axis_name="s")

    @functools.partial(
        pl.kernel,
        mesh=mesh,
        out_type=jax.ShapeDtypeStruct(
            (hist, n_dblk, n_bblk, 8, LANE), jnp.float32
        ),
        scratch_types=[
            pltpu.VMEM((LANE,), jnp.int32),
            pltpu.VMEM((LANE,), jnp.int32),
            pltpu.VMEM((LANE,), jnp.int32),
            pltpu.VMEM((LANE,), jnp.int32),
            pltpu.VMEM((LANE, GRP * emb_dim), jnp.float32),
            pltpu.VMEM((LANE, GRP * emb_dim), jnp.float32),
            pltpu.VMEM((emb_dim, LANE), jnp.float32),
            pltpu.VMEM((emb_dim, LANE), jnp.float32),
            pltpu.SemaphoreType.DMA,
            pltpu.SemaphoreType.DMA,
            pltpu.SemaphoreType.DMA,
            pltpu.SemaphoreType.DMA,
        ],
        compiler_params=pltpu.CompilerParams(
            use_tc_tiling_on_sc=False, needs_layout_passes=False
        ),
    )
    def body(hi_hbm, off_hbm, table_hbm, out_hbm,
             hi_a, hi_b, off_a, off_b, grp_a, grp_b, pl_a, pl_b,
             sga, sgb, soa, sob):
        w = lax.axis_index("s") * NUM_CORES + lax.axis_index("c")
        slots = ((hi_a, off_a, grp_a, pl_a, sga, soa),
                 (hi_b, off_b, grp_b, pl_b, sgb, sob))

        def stage(h, slot):
            hi_v, off_v, grp_v, _, sg, _ = slot
            pltpu.sync_copy(hi_hbm.at[h // 8, w, h % 8], hi_v)
            pltpu.sync_copy(off_hbm.at[h // 8, w, h % 8], off_v)
            pltpu.async_copy(table_hbm.at[hi_v], grp_v, sg)

        def gather_wait(slot):
            hi_v, _, grp_v, _, sg, _ = slot
            pltpu.make_async_copy(table_hbm.at[hi_v], grp_v, sg).wait()

        def put(h, slot):
            _, _, _, pl_v, _, so = slot
            for g in range(n_dblk):
                pltpu.async_copy(
                    pl_v.at[pl.ds(8 * g, 8), :], out_hbm.at[h, g, w], so
                )

        def put_wait(h, slot):
            _, _, _, pl_v, _, so = slot
            for g in range(n_dblk):
                pltpu.make_async_copy(
                    pl_v.at[pl.ds(8 * g, 8), :], out_hbm.at[h, g, w], so
                ).wait()

        def shuffle(slot):
            _, off_v, grp_v, pl_v, _, _ = slot
            base = lax.iota(jnp.int32, 16)
            cols = [off_v[pl.ds(j0 * 16, 16)] for j0 in range(LANE // 16)]
            rows = [base + (j0 * 16) for j0 in range(LANE // 16)]

            @plsc.parallel_loop(0, emb_dim, unroll=4)
            def _(d):
                for j0 in range(LANE // 16):
                    v = plsc.load_gather(grp_v, [rows[j0], cols[j0] + d])
                    pl_v[d, pl.ds(j0 * 16, 16)] = v

        stage(0, slots[0])

        def outer(o, carry):
            for b in (0, 1):
                h = o * 2 + b
                nxt = h + 1

                @pl.when(nxt < hist)
                def _():
                    stage(nxt, slots[1 - b])

                gather_wait(slots[b])

                @pl.when(h >= 2)
                def _():
                    put_wait(h - 2, slots[b])

                shuffle(slots[b])
                put(h, slots[b])
            return carry

        lax.fori_loop(0, hist // 2, outer, 0)
        put_wait(hist - 2, slots[0])
        put_wait(hist - 1, slots[1])

    return body


def kernel(indices, table):
    batch, hist = indices.shape
    vocab, emb_dim = table.shape
    n_bblk = batch // LANE
    hist_pad = -(-hist // 8) * 8
    n_hblk = hist_pad // 8
    n_cols = vocab // LANE
    idx_p = jnp.pad(indices.astype(jnp.int32), ((0, 0), (0, hist_pad - hist)))

    def to_tiles(a):
        return a.T.reshape(n_hblk, 8, n_bblk, LANE).transpose(0, 2, 1, 3)

    t0 = 999936
    n_main = t0 // GRP
    tail_j = idx_p - t0
    hi = jnp.where(idx_p < t0, idx_p % n_main, n_main + tail_j // GRP)
    off = jnp.where(idx_p < t0, (idx_p // n_main) * emb_dim,
                    (tail_j % GRP) * emb_dim)
    hi4 = to_tiles(hi)
    off4 = to_tiles(off)
    table_t = table.T
    tail16 = table[t0:].reshape((vocab - t0) // GRP, LANE)
    t2 = _tc_detile(vocab, emb_dim)(table_t, table_t, table_t, table_t, tail16)
    out5 = _emb_lookup(hist, emb_dim, n_bblk)(hi4, off4, t2)
    out = out5.transpose(2, 4, 0, 1, 3).reshape(batch, hist, emb_dim)
    return out


# speedup vs baseline: 1.8471x; 1.8471x over previous
import functools

import jax
import jax.numpy as jnp
from jax import lax
from jax.experimental import pallas as pl
from jax.experimental.pallas import tpu as pltpu
from jax.experimental.pallas import tpu_sc as plsc

NUM_CORES = 2
NUM_SUBCORES = 16
NUM_WORKERS = NUM_CORES * NUM_SUBCORES
LANE = 128
GRP = 4


def _emb_lookup(hist: int, emb_dim: int, n_bblk: int):
    n_dblk = emb_dim // 8
    mesh = plsc.VectorSubcoreMesh(core_axis_name="c", subcore_axis_name="s")

    @functools.partial(
        pl.kernel,
        mesh=mesh,
        out_type=jax.ShapeDtypeStruct(
            (hist, n_dblk, n_bblk, 8, LANE), jnp.float32
        ),
        scratch_types=[
            pltpu.VMEM((LANE,), jnp.int32),
            pltpu.VMEM((LANE,), jnp.int32),
            pltpu.VMEM((LANE,), jnp.int32),
            pltpu.VMEM((LANE,), jnp.int32),
            pltpu.VMEM((LANE, GRP * emb_dim), jnp.float32),
            pltpu.VMEM((LANE, GRP * emb_dim), jnp.float32),
            pltpu.VMEM((emb_dim, LANE), jnp.float32),
            pltpu.VMEM((emb_dim, LANE), jnp.float32),
            pltpu.SemaphoreType.DMA,
            pltpu.SemaphoreType.DMA,
            pltpu.SemaphoreType.DMA,
            pltpu.SemaphoreType.DMA,
        ],
        compiler_params=pltpu.CompilerParams(
            use_tc_tiling_on_sc=False, needs_layout_passes=False
        ),
    )
    def body(hi_hbm, off_hbm, table_hbm, out_hbm,
             hi_a, hi_b, off_a, off_b, grp_a, grp_b, pl_a, pl_b,
             sga, sgb, soa, sob):
        w = lax.axis_index("s") * NUM_CORES + lax.axis_index("c")
        slots = ((hi_a, off_a, grp_a, pl_a, sga, soa),
                 (hi_b, off_b, grp_b, pl_b, sgb, sob))

        def stage(h, slot):
            hi_v, off_v, grp_v, _, sg, _ = slot
            pltpu.sync_copy(hi_hbm.at[h // 8, w, h % 8], hi_v)
            pltpu.sync_copy(off_hbm.at[h // 8, w, h % 8], off_v)
            pltpu.async_copy(table_hbm.at[hi_v], grp_v, sg)

        def gather_wait(slot):
            hi_v, _, grp_v, _, sg, _ = slot
            pltpu.make_async_copy(table_hbm.at[hi_v], grp_v, sg).wait()

        def put(h, slot):
            _, _, _, pl_v, _, so = slot
            for g in range(n_dblk):
                pltpu.async_copy(
                    pl_v.at[pl.ds(8 * g, 8), :], out_hbm.at[h, g, w], so
                )

        def put_wait(h, slot):
            _, _, _, pl_v, _, so = slot
            for g in range(n_dblk):
                pltpu.make_async_copy(
                    pl_v.at[pl.ds(8 * g, 8), :], out_hbm.at[h, g, w], so
                ).wait()

        def shuffle(slot):
            _, off_v, grp_v, pl_v, _, _ = slot
            base = lax.iota(jnp.int32, 16)
            cols = [off_v[pl.ds(j0 * 16, 16)] for j0 in range(LANE // 16)]
            rows = [base + (j0 * 16) for j0 in range(LANE // 16)]

            @plsc.parallel_loop(0, emb_dim, unroll=4)
            def _(d):
                for j0 in range(LANE // 16):
                    v = plsc.load_gather(grp_v, [rows[j0], cols[j0] + d])
                    pl_v[d, pl.ds(j0 * 16, 16)] = v

        stage(0, slots[0])

        def outer(o, carry):
            for b in (0, 1):
                h = o * 2 + b
                nxt = h + 1

                @pl.when(nxt < hist)
                def _():
                    stage(nxt, slots[1 - b])

                gather_wait(slots[b])

                @pl.when(h >= 2)
                def _():
                    put_wait(h - 2, slots[b])

                shuffle(slots[b])
                put(h, slots[b])
            return carry

        lax.fori_loop(0, hist // 2, outer, 0)
        put_wait(hist - 2, slots[0])
        put_wait(hist - 1, slots[1])

    return body


def kernel(indices, table):
    batch, hist = indices.shape
    vocab, emb_dim = table.shape
    n_bblk = batch // LANE
    hist_pad = -(-hist // 8) * 8
    n_hblk = hist_pad // 8
    n_cols = vocab // LANE
    idx_p = jnp.pad(indices.astype(jnp.int32), ((0, 0), (0, hist_pad - hist)))

    def to_tiles(a):
        return a.T.reshape(n_hblk, 8, n_bblk, LANE).transpose(0, 2, 1, 3)

    hi4 = to_tiles(idx_p >> 1)
    off4 = to_tiles((idx_p & 1) << 5)
    t2 = jnp.pad(table.reshape(vocab // 2, 2 * emb_dim),
                 ((0, 0), (0, LANE - 2 * emb_dim)))
    out5 = _emb_lookup(hist, emb_dim, n_bblk)(hi4, off4, t2)
    out = out5.transpose(2, 4, 0, 1, 3).reshape(batch, hist, emb_dim)
    return out


# speedup vs baseline: 2.6148x; 1.4156x over previous
import functools

import jax
import jax.numpy as jnp
from jax import lax
from jax.experimental import pallas as pl
from jax.experimental.pallas import tpu as pltpu
from jax.experimental.pallas import tpu_sc as plsc

NUM_CORES = 2
NUM_SUBCORES = 16
NUM_WORKERS = NUM_CORES * NUM_SUBCORES
LANE = 128


def _emb_lookup(hist: int, emb_dim: int, n_bblk: int):
    n_dblk = emb_dim // 8
    mesh = plsc.VectorSubcoreMesh(core_axis_name="c", subcore_axis_name="s")

    @functools.partial(
        pl.kernel,
        mesh=mesh,
        out_type=jax.ShapeDtypeStruct(
            (hist, n_dblk, n_bblk, 8, LANE), jnp.float32
        ),
        scratch_types=[
            pltpu.VMEM((LANE,), jnp.int32),
            pltpu.VMEM((LANE,), jnp.int32),
            pltpu.VMEM((LANE, LANE), jnp.float32),
            pltpu.VMEM((LANE, LANE), jnp.float32),
            pltpu.VMEM((emb_dim, LANE), jnp.float32),
            pltpu.VMEM((emb_dim, LANE), jnp.float32),
            pltpu.SemaphoreType.DMA,
            pltpu.SemaphoreType.DMA,
            pltpu.SemaphoreType.DMA,
            pltpu.SemaphoreType.DMA,
        ],
        compiler_params=pltpu.CompilerParams(
            use_tc_tiling_on_sc=False, needs_layout_passes=False
        ),
    )
    def body(idx_hbm, table_hbm, out_hbm,
             idx_a, idx_b, grp_a, grp_b, pl_a, pl_b, sga, sgb, soa, sob):
        w = lax.axis_index("s") * NUM_CORES + lax.axis_index("c")
        slots = ((idx_a, grp_a, pl_a, sga, soa),
                 (idx_b, grp_b, pl_b, sgb, sob))

        def stage(h, slot):
            idx_v, grp_v, _, sg, _ = slot
            pltpu.sync_copy(idx_hbm.at[h // 8, w, h % 8], idx_v)
            pltpu.async_copy(table_hbm.at[idx_v], grp_v, sg)

        def gather_wait(slot):
            idx_v, grp_v, _, sg, _ = slot
            pltpu.make_async_copy(table_hbm.at[idx_v], grp_v, sg).wait()

        def put(h, slot):
            _, _, pl_v, _, so = slot
            for g in range(n_dblk):
                pltpu.async_copy(
                    pl_v.at[pl.ds(8 * g, 8), :], out_hbm.at[h, g, w], so
                )

        def put_wait(h, slot):
            _, _, pl_v, _, so = slot
            for g in range(n_dblk):
                pltpu.make_async_copy(
                    pl_v.at[pl.ds(8 * g, 8), :], out_hbm.at[h, g, w], so
                ).wait()

        def shuffle(slot):
            _, grp_v, pl_v, _, _ = slot
            base = lax.iota(jnp.int32, 16)
            rows = [base + (j0 * 16) for j0 in range(LANE // 16)]

            @plsc.parallel_loop(0, emb_dim, unroll=4)
            def _(d):
                col = jnp.full((16,), d, jnp.int32)
                for j0 in range(LANE // 16):
                    v = plsc.load_gather(grp_v, [rows[j0], col])
                    pl_v[d, pl.ds(j0 * 16, 16)] = v

        stage(0, slots[0])

        def outer(o, carry):
            for b in (0, 1):
                h = o * 2 + b
                nxt = h + 1

                @pl.when(nxt < hist)
                def _():
                    stage(nxt, slots[1 - b])

                gather_wait(slots[b])

                @pl.when(h >= 2)
                def _():
                    put_wait(h - 2, slots[b])

                shuffle(slots[b])
                put(h, slots[b])
            return carry

        lax.fori_loop(0, hist // 2, outer, 0)
        put_wait(hist - 2, slots[0])
        put_wait(hist - 1, slots[1])

    return body


def kernel(indices, table):
    batch, hist = indices.shape
    vocab, emb_dim = table.shape
    n_bblk = batch // LANE
    hist_pad = -(-hist // 8) * 8
    n_hblk = hist_pad // 8
    idx_p = jnp.pad(indices.astype(jnp.int32), ((0, 0), (0, hist_pad - hist)))
    idx4 = idx_p.T.reshape(n_hblk, 8, n_bblk, LANE).transpose(0, 2, 1, 3)
    t2 = jnp.pad(table, ((0, 0), (0, LANE - emb_dim)))
    out5 = _emb_lookup(hist, emb_dim, n_bblk)(idx4, t2)
    out = out5.transpose(2, 4, 0, 1, 3).reshape(batch, hist, emb_dim)
    return out
